# baseline (speedup 1.0000x reference)
import functools

import jax
import jax.numpy as jnp
from jax import lax
from jax.experimental import pallas as pl
from jax.experimental.pallas import tpu as pltpu
from jax.experimental.pallas import tpu_sc as plsc

_CHUNK = 128


@functools.cache
def _make_gather(B, V, D):
    info = plsc.get_sparse_core_info()
    NC, NS = info.num_cores, info.num_subcores
    NW = NC * NS
    b_per_w = B // NW
    n_chunks = b_per_w // _CHUNK
    mesh = plsc.VectorSubcoreMesh(core_axis_name="c", subcore_axis_name="s")

    @functools.partial(
        pl.kernel,
        mesh=mesh,
        out_type=jax.ShapeDtypeStruct((B, D), jnp.float32),
        scratch_types=[
            pltpu.VMEM((n_chunks, _CHUNK), jnp.int32),
            pltpu.VMEM((b_per_w, D), jnp.float32),
            pltpu.SemaphoreType.DMA,
        ],
        compiler_params=pltpu.CompilerParams(use_tc_tiling_on_sc=False),
    )
    def gather_kernel(table_hbm, idx_hbm, out_hbm, idx_v, rows_v, sem):
        wid = lax.axis_index("s") * NC + lax.axis_index("c")
        base = wid * b_per_w
        pltpu.sync_copy(idx_hbm.at[pl.ds(wid * n_chunks, n_chunks)], idx_v)
        copies = [
            pltpu.async_copy(
                table_hbm.at[idx_v.at[j]],
                rows_v.at[pl.ds(j * _CHUNK, _CHUNK)],
                sem,
            )
            for j in range(n_chunks)
        ]
        for c in copies:
            c.wait()
        pltpu.sync_copy(rows_v, out_hbm.at[pl.ds(base, b_per_w)])

    return gather_kernel


def kernel(batch, table):
    B = batch.shape[0]
    V, D = table.shape
    idx2 = batch.reshape(B // _CHUNK, _CHUNK)
    out = _make_gather(B, V, D)(table, idx2)
    return out[:, None, :]


# speedup vs baseline: 1.3888x; 1.3888x over previous
import functools

import jax
import jax.numpy as jnp
from jax import lax
from jax.experimental import pallas as pl
from jax.experimental.pallas import tpu as pltpu
from jax.experimental.pallas import tpu_sc as plsc

_L = 16


@functools.cache
def _make_gather(B, V, D):
    info = plsc.get_sparse_core_info()
    NC, NS = info.num_cores, info.num_subcores
    NW = NC * NS
    b_per_w = B // NW
    n_groups = b_per_w // _L
    mesh = plsc.VectorSubcoreMesh(core_axis_name="c", subcore_axis_name="s")

    @functools.partial(
        pl.kernel,
        mesh=mesh,
        out_type=jax.ShapeDtypeStruct((B, D), jnp.float32),
        scratch_types=[
            pltpu.VMEM((b_per_w,), jnp.int32),
            pltpu.VMEM((b_per_w, D), jnp.float32),
            pltpu.SemaphoreType.DMA,
        ],
        compiler_params=pltpu.CompilerParams(needs_layout_passes=False),
    )
    def gather_kernel(table_hbm, idx_hbm, out_hbm, idx_v, rows_v, sem):
        wid = lax.axis_index("s") * NC + lax.axis_index("c")
        base = wid * b_per_w
        pltpu.sync_copy(idx_hbm.at[pl.ds(base, b_per_w)], idx_v)
        lane = lax.broadcasted_iota(jnp.int32, (_L,), 0)

        def body(g, carry):
            vec = idx_v[pl.ds(g * _L, _L)]
            for j in range(_L):
                row = jnp.sum(jnp.where(lane == j, vec, 0))
                pltpu.async_copy(
                    table_hbm.at[pl.ds(row, 1)],
                    rows_v.at[pl.ds(g * _L + j, 1)],
                    sem,
                )
            @pl.when(g >= 2)
            def _():
                pltpu.make_async_copy(
                    table_hbm.at[pl.ds(0, _L)], rows_v.at[pl.ds(0, _L)], sem
                ).wait()

            return carry

        lax.fori_loop(0, n_groups, body, 0)
        pltpu.make_async_copy(
            table_hbm.at[pl.ds(0, 2 * _L)], rows_v.at[pl.ds(0, 2 * _L)], sem
        ).wait()
        pltpu.sync_copy(rows_v, out_hbm.at[pl.ds(base, b_per_w)])

    return gather_kernel


def kernel(batch, table):
    B = batch.shape[0]
    V, D = table.shape
    out = _make_gather(B, V, D)(table, batch)
    return out[:, None, :]


# speedup vs baseline: 1.4672x; 1.0565x over previous
import functools

import jax
import jax.numpy as jnp
from jax import lax
from jax.experimental import pallas as pl
from jax.experimental.pallas import tpu as pltpu
from jax.experimental.pallas import tpu_sc as plsc

_L = 16
_DRAIN_EVERY = 4


@functools.cache
def _make_gather(B, V, D):
    info = plsc.get_sparse_core_info()
    NC, NS = info.num_cores, info.num_subcores
    NW = NC * NS
    b_per_w = B // NW
    n_groups = b_per_w // _L
    mesh = plsc.VectorSubcoreMesh(core_axis_name="c", subcore_axis_name="s")

    @functools.partial(
        pl.kernel,
        mesh=mesh,
        out_type=jax.ShapeDtypeStruct((B, D), jnp.float32),
        scratch_types=[
            pltpu.VMEM((b_per_w,), jnp.int32),
            pltpu.VMEM((b_per_w, D), jnp.float32),
            pltpu.SemaphoreType.DMA,
        ],
        compiler_params=pltpu.CompilerParams(needs_layout_passes=False),
    )
    def gather_kernel(table_hbm, idx_hbm, out_hbm, idx_v, rows_v, sem):
        wid = lax.axis_index("s") * NC + lax.axis_index("c")
        base = wid * b_per_w
        pltpu.sync_copy(idx_hbm.at[pl.ds(base, b_per_w)], idx_v)
        lane = lax.broadcasted_iota(jnp.int32, (_L,), 0)

        def body(g, carry):
            vec = idx_v[pl.ds(g * _L, _L)]
            for j in range(_L):
                row = jnp.sum(jnp.where(lane == j, vec, 0))
                pltpu.async_copy(
                    table_hbm.at[pl.ds(row, 1)],
                    rows_v.at[pl.ds(g * _L + j, 1)],
                    sem,
                )
            @pl.when(jnp.logical_and(g >= 2 * _DRAIN_EVERY - 1,
                                     g % _DRAIN_EVERY == _DRAIN_EVERY - 1))
            def _():
                pltpu.make_async_copy(
                    table_hbm.at[pl.ds(0, _DRAIN_EVERY * _L)],
                    rows_v.at[pl.ds(0, _DRAIN_EVERY * _L)],
                    sem,
                ).wait()

            return carry

        lax.fori_loop(0, n_groups, body, 0, unroll=2)
        pltpu.make_async_copy(
            table_hbm.at[pl.ds(0, _DRAIN_EVERY * _L)],
            rows_v.at[pl.ds(0, _DRAIN_EVERY * _L)],
            sem,
        ).wait()
        pltpu.sync_copy(rows_v, out_hbm.at[pl.ds(base, b_per_w)])

    return gather_kernel


def kernel(batch, table):
    B = batch.shape[0]
    V, D = table.shape
    out = _make_gather(B, V, D)(table, batch)
    return out[:, None, :]
